# baseline (speedup 1.0000x reference)
import functools

import jax
import jax.numpy as jnp
from jax import lax
from jax.experimental import pallas as pl
from jax.experimental.pallas import tpu as pltpu
from jax.experimental.pallas import tpu_sc as plsc

_K = 16
_LANES = 16
_NC = 2
_NS = 16
_PAD = 256
_UNROLL = 4


def _sc_scatter_body(L, p_hbm, out_hbm, p_v, wm_a, wm_b, hc_a, hc_b):
    n_chunks = L // (_NC * _NS * _LANES)
    cid = lax.axis_index("c")
    sid = lax.axis_index("s")
    wid = cid * _NS + sid

    pltpu.sync_copy(p_hbm, p_v)

    iota_i = jax.lax.iota(jnp.int32, _LANES)
    iota_f = iota_i.astype(jnp.float32)
    zf = jnp.zeros((_LANES,), jnp.float32)
    dvec = p_v[2 * _K]

    Lp = L + _PAD

    def zbody(i, _):
        off = i * (4 * _LANES)
        for u in range(4):
            wm_a[pl.ds(off + u * _LANES, _LANES)] = zf
            wm_b[pl.ds(off + u * _LANES, _LANES)] = zf
            hc_a[pl.ds(off + u * _LANES, _LANES)] = zf
            hc_b[pl.ds(off + u * _LANES, _LANES)] = zf
        return 0

    lax.fori_loop(0, Lp // (4 * _LANES), zbody, 0, unroll=False)

    l_base = (wid * (L // (_NC * _NS))).astype(jnp.float32)
    fmax = float(L - 1)

    blo_w = blo_h = bhi_w = bhi_h = zf

    for k in range(_K):
        wk = p_v[k]
        rwk = p_v[_K + k]
        wm_v = wm_a if (k % 2 == 0) else wm_b
        hc_v = hc_a if (k % 2 == 0) else hc_b

        def chunk(lvec, c_lo, c_hi, wk=wk, rwk=rwk, wm_v=wm_v, hc_v=hc_v):
            centers = lvec + dvec
            p = centers + wk
            p_cl = jnp.minimum(jnp.maximum(p, 0.0), fmax)
            pf_i = p_cl.astype(jnp.int32)
            frac = p_cl - pf_i.astype(jnp.float32)
            omf = 1.0 - frac
            m_lo = p <= 0.0
            m_hi = p >= fmax
            m_in = jnp.logical_not(jnp.logical_or(m_lo, m_hi))
            pc_i = pf_i + 1
            plsc.addupdate_scatter(wm_v, [pf_i], omf * rwk, mask=m_in)
            plsc.addupdate_scatter(wm_v, [pc_i], frac * rwk, mask=m_in)
            plsc.addupdate_scatter(hc_v, [pf_i], omf, mask=m_in)
            plsc.addupdate_scatter(hc_v, [pc_i], frac, mask=m_in)
            c_lo = c_lo + jnp.where(m_lo, 1.0, 0.0)
            c_hi = c_hi + jnp.where(m_hi, 1.0, 0.0)
            return c_lo, c_hi

        def body(j, c, chunk=chunk):
            lvec, c_lo, c_hi = c
            for _ in range(_UNROLL):
                c_lo, c_hi = chunk(lvec, c_lo, c_hi)
                lvec = lvec + float(_LANES)
            return (lvec, c_lo, c_hi)

        _, c_lo, c_hi = lax.fori_loop(0, n_chunks // _UNROLL, body,
                                      (l_base + iota_f, zf, zf))
        blo_w = blo_w + c_lo * rwk
        blo_h = blo_h + c_lo
        bhi_w = bhi_w + c_hi * rwk
        bhi_h = bhi_h + c_hi

    col_lo = iota_i + L
    col_hi = iota_i + (L + 128)
    plsc.addupdate_scatter(wm_a, [col_lo], blo_w)
    plsc.addupdate_scatter(hc_a, [col_lo], blo_h)
    plsc.addupdate_scatter(wm_a, [col_hi], bhi_w)
    plsc.addupdate_scatter(hc_a, [col_hi], bhi_h)

    def mbody(i, _):
        off = i * (2 * _LANES)
        for u in range(2):
            o = off + u * _LANES
            wm_a[pl.ds(o, _LANES)] = wm_a[pl.ds(o, _LANES)] + wm_b[pl.ds(o, _LANES)]
            hc_a[pl.ds(o, _LANES)] = hc_a[pl.ds(o, _LANES)] + hc_b[pl.ds(o, _LANES)]
        return 0

    lax.fori_loop(0, Lp // (2 * _LANES), mbody, 0, unroll=False)

    pltpu.sync_copy(wm_a, out_hbm.at[wid, 0])
    pltpu.sync_copy(hc_a, out_hbm.at[wid, 1])


def _sc_scatter(L, params):
    Lp = L + _PAD
    mesh = plsc.VectorSubcoreMesh(core_axis_name="c", subcore_axis_name="s")
    fn = functools.partial(
        pl.kernel,
        out_type=jax.ShapeDtypeStruct((_NC * _NS, 2, Lp), jnp.float32),
        mesh=mesh,
        compiler_params=pltpu.CompilerParams(needs_layout_passes=False),
        scratch_types=[
            pltpu.VMEM((2 * _K + 1, _LANES), jnp.float32),
            pltpu.VMEM((Lp,), jnp.float32),
            pltpu.VMEM((Lp,), jnp.float32),
            pltpu.VMEM((Lp,), jnp.float32),
            pltpu.VMEM((Lp,), jnp.float32),
        ],
    )(functools.partial(_sc_scatter_body, L))
    return fn(params)


def _tc_norm_mul_body(L, parts_ref, x_ref, o_ref, norm_ref):
    @pl.when(pl.program_id(0) == 0)
    def _():
        s = jnp.sum(parts_ref[...], axis=0)
        pw = s[0:1, :]
        ph = s[1:2, :]
        b0w = jnp.sum(pw[:, L:L + 16])
        b0h = jnp.sum(ph[:, L:L + 16])
        bLw = jnp.sum(pw[:, L + 128:L + 144])
        bLh = jnp.sum(ph[:, L + 128:L + 144])
        col = lax.broadcasted_iota(jnp.int32, (1, L), 1)
        wm = pw[:, :L] + jnp.where(col == 0, b0w, 0.0) + jnp.where(col == L - 1, bLw, 0.0)
        hc = ph[:, :L] + jnp.where(col == 0, b0h, 0.0) + jnp.where(col == L - 1, bLh, 0.0)
        avg = wm / jnp.maximum(hc, 1e-6)
        norm_ref[...] = avg / jnp.maximum(jnp.sum(avg, axis=1, keepdims=True), 1e-6)

    o_ref[...] = x_ref[...] * norm_ref[...]


def _tc_norm_mul(x, parts):
    B, L = x.shape
    Lp = L + _PAD
    rows = 32
    grid = (B // rows,)
    return pl.pallas_call(
        functools.partial(_tc_norm_mul_body, L),
        grid=grid,
        in_specs=[
            pl.BlockSpec((_NC * _NS, 2, Lp), lambda i: (0, 0, 0)),
            pl.BlockSpec((rows, L), lambda i: (i, 0)),
        ],
        out_specs=pl.BlockSpec((rows, L), lambda i: (i, 0)),
        out_shape=jax.ShapeDtypeStruct((B, L), x.dtype),
        scratch_shapes=[pltpu.VMEM((1, L), jnp.float32)],
    )(parts, x)


def kernel(x, base_deformation, base_stride, base_beta_fwd, base_beta_bwd,
           base_strength, base_alpha_fwd, base_alpha_bwd, sample_bias):
    B, L = x.shape
    K = _K
    d = jnp.clip(base_deformation, -32.0, 32.0)
    stride = jax.nn.softplus(base_stride)
    beta_fwd = jax.nn.softplus(base_beta_fwd)
    beta_bwd = jax.nn.softplus(base_beta_bwd)
    strength = jax.nn.softplus(base_strength)
    alpha_fwd = jax.nn.softplus(base_alpha_fwd)
    alpha_bwd = jax.nn.softplus(base_alpha_bwd)
    k2 = lax.broadcasted_iota(jnp.float32, (K, _LANES), 0) - (K // 2)
    k_abs = jnp.abs(k2)
    w16 = jnp.where(k2 >= 0, (k_abs ** beta_fwd) * stride,
                    -((k_abs ** beta_bwd) * stride))
    envelope = jnp.where(k2 >= 0, strength / (1.0 + k_abs) ** alpha_fwd,
                         strength / (1.0 + k_abs) ** alpha_bwd)
    bias2 = lax.broadcast_in_dim(sample_bias, (K, _LANES), (0,))
    rw16 = envelope * (1.0 + jnp.tanh(bias2))
    d16 = lax.broadcast_in_dim(d, (1, _LANES), ())
    params = jnp.concatenate([w16, rw16, d16], axis=0)

    parts = _sc_scatter(L, params)
    return _tc_norm_mul(x, parts)


# speedup vs baseline: 1.0179x; 1.0179x over previous
import functools

import jax
import jax.numpy as jnp
from jax import lax
from jax.experimental import pallas as pl
from jax.experimental.pallas import tpu as pltpu
from jax.experimental.pallas import tpu_sc as plsc

_K = 16
_LANES = 16
_NC = 2
_NS = 16
_PAD = 256
_UNROLL = 4


def _sc_scatter_body(L, d_hbm, w_hbm, rw_hbm, out_hbm, d_v, w_v, rw_v,
                     wm_v, hc_v):
    n_chunks = L // (_NC * _NS * _LANES)
    cid = lax.axis_index("c")
    sid = lax.axis_index("s")
    wid = cid * _NS + sid

    pltpu.sync_copy(d_hbm, d_v)
    pltpu.sync_copy(w_hbm, w_v)
    pltpu.sync_copy(rw_hbm, rw_v)

    iota_i = jax.lax.iota(jnp.int32, _LANES)
    iota_f = iota_i.astype(jnp.float32)
    zf = jnp.zeros((_LANES,), jnp.float32)
    dvec = d_v[...]

    Lp = L + _PAD

    def zbody(i, _):
        off = i * (4 * _LANES)
        for u in range(4):
            wm_v[pl.ds(off + u * _LANES, _LANES)] = zf
            hc_v[pl.ds(off + u * _LANES, _LANES)] = zf
        return 0

    lax.fori_loop(0, Lp // (4 * _LANES), zbody, 0, unroll=False)

    l_base = (wid * (L // (_NC * _NS))).astype(jnp.float32)
    fmax = float(L - 1)

    blo_w = blo_h = bhi_w = bhi_h = zf

    def chunk(lvec, c_lo, c_hi, wk, rwk):
        centers = lvec + dvec
        p = centers + wk
        p_cl = jnp.minimum(jnp.maximum(p, 0.0), fmax)
        pf_i = p_cl.astype(jnp.int32)
        frac = p_cl - pf_i.astype(jnp.float32)
        omf = 1.0 - frac
        m_lo = p <= 0.0
        m_hi = p >= fmax
        m_in = jnp.logical_not(jnp.logical_or(m_lo, m_hi))
        pc_i = pf_i + 1
        plsc.addupdate_scatter(wm_v, [pf_i], omf * rwk, mask=m_in)
        plsc.addupdate_scatter(wm_v, [pc_i], frac * rwk, mask=m_in)
        plsc.addupdate_scatter(hc_v, [pf_i], omf, mask=m_in)
        plsc.addupdate_scatter(hc_v, [pc_i], frac, mask=m_in)
        c_lo = c_lo + jnp.where(m_lo, 1.0, 0.0)
        c_hi = c_hi + jnp.where(m_hi, 1.0, 0.0)
        return c_lo, c_hi

    for k in range(0, _K, 2):
        wk0, rwk0 = w_v[k], rw_v[k]
        wk1, rwk1 = w_v[k + 1], rw_v[k + 1]

        def body(j, c, wk0=wk0, rwk0=rwk0, wk1=wk1, rwk1=rwk1):
            lvec, c0_lo, c0_hi, c1_lo, c1_hi = c
            for _ in range(_UNROLL):
                c0_lo, c0_hi = chunk(lvec, c0_lo, c0_hi, wk0, rwk0)
                c1_lo, c1_hi = chunk(lvec, c1_lo, c1_hi, wk1, rwk1)
                lvec = lvec + float(_LANES)
            return (lvec, c0_lo, c0_hi, c1_lo, c1_hi)

        _, c0_lo, c0_hi, c1_lo, c1_hi = lax.fori_loop(
            0, n_chunks // _UNROLL, body,
            (l_base + iota_f, zf, zf, zf, zf))
        blo_w = blo_w + c0_lo * rwk0 + c1_lo * rwk1
        blo_h = blo_h + c0_lo + c1_lo
        bhi_w = bhi_w + c0_hi * rwk0 + c1_hi * rwk1
        bhi_h = bhi_h + c0_hi + c1_hi

    col_lo = iota_i + L
    col_hi = iota_i + (L + 128)
    plsc.addupdate_scatter(wm_v, [col_lo], blo_w)
    plsc.addupdate_scatter(hc_v, [col_lo], blo_h)
    plsc.addupdate_scatter(wm_v, [col_hi], bhi_w)
    plsc.addupdate_scatter(hc_v, [col_hi], bhi_h)

    pltpu.sync_copy(wm_v, out_hbm.at[wid, 0])
    pltpu.sync_copy(hc_v, out_hbm.at[wid, 1])


def _sc_scatter(L, d16, w16, rw16):
    Lp = L + _PAD
    mesh = plsc.VectorSubcoreMesh(core_axis_name="c", subcore_axis_name="s")
    fn = functools.partial(
        pl.kernel,
        out_type=jax.ShapeDtypeStruct((_NC * _NS, 2, Lp), jnp.float32),
        mesh=mesh,
        compiler_params=pltpu.CompilerParams(needs_layout_passes=False),
        scratch_types=[
            pltpu.VMEM((_LANES,), jnp.float32),
            pltpu.VMEM((_K, _LANES), jnp.float32),
            pltpu.VMEM((_K, _LANES), jnp.float32),
            pltpu.VMEM((Lp,), jnp.float32),
            pltpu.VMEM((Lp,), jnp.float32),
        ],
    )(functools.partial(_sc_scatter_body, L))
    return fn(d16, w16, rw16)


def _tc_norm_mul_body(L, parts_ref, x_ref, o_ref, norm_ref):
    @pl.when(pl.program_id(0) == 0)
    def _():
        s = jnp.sum(parts_ref[...], axis=0)
        pw = s[0:1, :]
        ph = s[1:2, :]
        b0w = jnp.sum(pw[:, L:L + 16])
        b0h = jnp.sum(ph[:, L:L + 16])
        bLw = jnp.sum(pw[:, L + 128:L + 144])
        bLh = jnp.sum(ph[:, L + 128:L + 144])
        col = lax.broadcasted_iota(jnp.int32, (1, L), 1)
        wm = pw[:, :L] + jnp.where(col == 0, b0w, 0.0) + jnp.where(col == L - 1, bLw, 0.0)
        hc = ph[:, :L] + jnp.where(col == 0, b0h, 0.0) + jnp.where(col == L - 1, bLh, 0.0)
        avg = wm / jnp.maximum(hc, 1e-6)
        norm_ref[...] = avg / jnp.maximum(jnp.sum(avg, axis=1, keepdims=True), 1e-6)

    o_ref[...] = x_ref[...] * norm_ref[...]


def _tc_norm_mul(x, parts):
    B, L = x.shape
    Lp = L + _PAD
    rows = 32
    grid = (B // rows,)
    return pl.pallas_call(
        functools.partial(_tc_norm_mul_body, L),
        grid=grid,
        in_specs=[
            pl.BlockSpec((_NC * _NS, 2, Lp), lambda i: (0, 0, 0)),
            pl.BlockSpec((rows, L), lambda i: (i, 0)),
        ],
        out_specs=pl.BlockSpec((rows, L), lambda i: (i, 0)),
        out_shape=jax.ShapeDtypeStruct((B, L), x.dtype),
        scratch_shapes=[pltpu.VMEM((1, L), jnp.float32)],
    )(parts, x)


def kernel(x, base_deformation, base_stride, base_beta_fwd, base_beta_bwd,
           base_strength, base_alpha_fwd, base_alpha_bwd, sample_bias):
    B, L = x.shape
    K = _K
    d = jnp.clip(base_deformation, -32.0, 32.0)
    stride = jax.nn.softplus(base_stride)
    beta_fwd = jax.nn.softplus(base_beta_fwd)
    beta_bwd = jax.nn.softplus(base_beta_bwd)
    strength = jax.nn.softplus(base_strength)
    alpha_fwd = jax.nn.softplus(base_alpha_fwd)
    alpha_bwd = jax.nn.softplus(base_alpha_bwd)
    k2 = lax.broadcasted_iota(jnp.float32, (K, _LANES), 0) - (K // 2)
    k_abs = jnp.abs(k2)
    w16 = jnp.where(k2 >= 0, (k_abs ** beta_fwd) * stride,
                    -((k_abs ** beta_bwd) * stride))
    envelope = jnp.where(k2 >= 0, strength / (1.0 + k_abs) ** alpha_fwd,
                         strength / (1.0 + k_abs) ** alpha_bwd)
    bias2 = lax.broadcast_in_dim(sample_bias, (K, _LANES), (0,))
    rw16 = envelope * (1.0 + jnp.tanh(bias2))
    d16 = lax.broadcast_in_dim(d, (_LANES,), ())

    parts = _sc_scatter(L, d16, w16, rw16)
    return _tc_norm_mul(x, parts)


# speedup vs baseline: 1.0389x; 1.0206x over previous
import functools

import jax
import jax.numpy as jnp
from jax import lax
from jax.experimental import pallas as pl
from jax.experimental.pallas import tpu as pltpu
from jax.experimental.pallas import tpu_sc as plsc

_K = 16
_LANES = 16
_NC = 2
_NS = 16
_PAD = 256
_UNROLL = 2


def _sc_scatter_body(L, d_hbm, w_hbm, rw_hbm, out_hbm, d_v, w_v, rw_v,
                     wm_v, hc_v):
    n_chunks = L // (_NC * _NS * _LANES)
    cid = lax.axis_index("c")
    sid = lax.axis_index("s")
    wid = cid * _NS + sid

    pltpu.sync_copy(d_hbm, d_v)
    pltpu.sync_copy(w_hbm, w_v)
    pltpu.sync_copy(rw_hbm, rw_v)

    iota_i = jax.lax.iota(jnp.int32, _LANES)
    iota_f = iota_i.astype(jnp.float32)
    zf = jnp.zeros((_LANES,), jnp.float32)
    dvec = d_v[...]

    Lp = L + _PAD

    def zbody(i, _):
        off = i * (4 * _LANES)
        for u in range(4):
            wm_v[pl.ds(off + u * _LANES, _LANES)] = zf
            hc_v[pl.ds(off + u * _LANES, _LANES)] = zf
        return 0

    lax.fori_loop(0, Lp // (4 * _LANES), zbody, 0, unroll=False)

    l_base = (wid * (L // (_NC * _NS))).astype(jnp.float32)
    fmax = float(L - 1)

    blo_w = blo_h = bhi_w = bhi_h = zf

    def chunk(lvec, c_lo, c_hi, wk, rwk):
        centers = lvec + dvec
        p = centers + wk
        p_cl = jnp.minimum(jnp.maximum(p, 0.0), fmax)
        pf_i = p_cl.astype(jnp.int32)
        frac = p_cl - pf_i.astype(jnp.float32)
        omf = 1.0 - frac
        m_lo = p <= 0.0
        m_hi = p >= fmax
        m_in = jnp.logical_not(jnp.logical_or(m_lo, m_hi))
        pc_i = pf_i + 1
        plsc.addupdate_scatter(wm_v, [pf_i], omf * rwk, mask=m_in)
        plsc.addupdate_scatter(wm_v, [pc_i], frac * rwk, mask=m_in)
        plsc.addupdate_scatter(hc_v, [pf_i], omf, mask=m_in)
        plsc.addupdate_scatter(hc_v, [pc_i], frac, mask=m_in)
        c_lo = c_lo + jnp.where(m_lo, 1.0, 0.0)
        c_hi = c_hi + jnp.where(m_hi, 1.0, 0.0)
        return c_lo, c_hi

    _GRP = 4
    for k in range(0, _K, _GRP):
        wks = [w_v[k + g] for g in range(_GRP)]
        rwks = [rw_v[k + g] for g in range(_GRP)]

        def body(j, c, wks=wks, rwks=rwks):
            lvec = c[0]
            cs = list(c[1:])
            for _ in range(_UNROLL):
                for g in range(_GRP):
                    cs[2 * g], cs[2 * g + 1] = chunk(
                        lvec, cs[2 * g], cs[2 * g + 1], wks[g], rwks[g])
                lvec = lvec + float(_LANES)
            return (lvec, *cs)

        out_c = lax.fori_loop(0, n_chunks // _UNROLL, body,
                              ((l_base + iota_f),) + tuple([zf] * (2 * _GRP)))
        for g in range(_GRP):
            c_lo, c_hi = out_c[1 + 2 * g], out_c[2 + 2 * g]
            blo_w = blo_w + c_lo * rwks[g]
            blo_h = blo_h + c_lo
            bhi_w = bhi_w + c_hi * rwks[g]
            bhi_h = bhi_h + c_hi

    col_lo = iota_i + L
    col_hi = iota_i + (L + 128)
    plsc.addupdate_scatter(wm_v, [col_lo], blo_w)
    plsc.addupdate_scatter(hc_v, [col_lo], blo_h)
    plsc.addupdate_scatter(wm_v, [col_hi], bhi_w)
    plsc.addupdate_scatter(hc_v, [col_hi], bhi_h)

    pltpu.sync_copy(wm_v, out_hbm.at[wid, 0])
    pltpu.sync_copy(hc_v, out_hbm.at[wid, 1])


def _sc_scatter(L, d16, w16, rw16):
    Lp = L + _PAD
    mesh = plsc.VectorSubcoreMesh(core_axis_name="c", subcore_axis_name="s")
    fn = functools.partial(
        pl.kernel,
        out_type=jax.ShapeDtypeStruct((_NC * _NS, 2, Lp), jnp.float32),
        mesh=mesh,
        compiler_params=pltpu.CompilerParams(needs_layout_passes=False),
        scratch_types=[
            pltpu.VMEM((_LANES,), jnp.float32),
            pltpu.VMEM((_K, _LANES), jnp.float32),
            pltpu.VMEM((_K, _LANES), jnp.float32),
            pltpu.VMEM((Lp,), jnp.float32),
            pltpu.VMEM((Lp,), jnp.float32),
        ],
    )(functools.partial(_sc_scatter_body, L))
    return fn(d16, w16, rw16)


def _tc_norm_mul_body(L, parts_ref, x_ref, o_ref, norm_ref):
    @pl.when(pl.program_id(0) == 0)
    def _():
        s = jnp.sum(parts_ref[...], axis=0)
        pw = s[0:1, :]
        ph = s[1:2, :]
        b0w = jnp.sum(pw[:, L:L + 16])
        b0h = jnp.sum(ph[:, L:L + 16])
        bLw = jnp.sum(pw[:, L + 128:L + 144])
        bLh = jnp.sum(ph[:, L + 128:L + 144])
        col = lax.broadcasted_iota(jnp.int32, (1, L), 1)
        wm = pw[:, :L] + jnp.where(col == 0, b0w, 0.0) + jnp.where(col == L - 1, bLw, 0.0)
        hc = ph[:, :L] + jnp.where(col == 0, b0h, 0.0) + jnp.where(col == L - 1, bLh, 0.0)
        avg = wm / jnp.maximum(hc, 1e-6)
        norm_ref[...] = avg / jnp.maximum(jnp.sum(avg, axis=1, keepdims=True), 1e-6)

    o_ref[...] = x_ref[...] * norm_ref[...]


def _tc_norm_mul(x, parts):
    B, L = x.shape
    Lp = L + _PAD
    rows = 32
    grid = (B // rows,)
    return pl.pallas_call(
        functools.partial(_tc_norm_mul_body, L),
        grid=grid,
        in_specs=[
            pl.BlockSpec((_NC * _NS, 2, Lp), lambda i: (0, 0, 0)),
            pl.BlockSpec((rows, L), lambda i: (i, 0)),
        ],
        out_specs=pl.BlockSpec((rows, L), lambda i: (i, 0)),
        out_shape=jax.ShapeDtypeStruct((B, L), x.dtype),
        scratch_shapes=[pltpu.VMEM((1, L), jnp.float32)],
    )(parts, x)


def kernel(x, base_deformation, base_stride, base_beta_fwd, base_beta_bwd,
           base_strength, base_alpha_fwd, base_alpha_bwd, sample_bias):
    B, L = x.shape
    K = _K
    d = jnp.clip(base_deformation, -32.0, 32.0)
    stride = jax.nn.softplus(base_stride)
    beta_fwd = jax.nn.softplus(base_beta_fwd)
    beta_bwd = jax.nn.softplus(base_beta_bwd)
    strength = jax.nn.softplus(base_strength)
    alpha_fwd = jax.nn.softplus(base_alpha_fwd)
    alpha_bwd = jax.nn.softplus(base_alpha_bwd)
    k2 = lax.broadcasted_iota(jnp.float32, (K, _LANES), 0) - (K // 2)
    k_abs = jnp.abs(k2)
    w16 = jnp.where(k2 >= 0, (k_abs ** beta_fwd) * stride,
                    -((k_abs ** beta_bwd) * stride))
    envelope = jnp.where(k2 >= 0, strength / (1.0 + k_abs) ** alpha_fwd,
                         strength / (1.0 + k_abs) ** alpha_bwd)
    bias2 = lax.broadcast_in_dim(sample_bias, (K, _LANES), (0,))
    rw16 = envelope * (1.0 + jnp.tanh(bias2))
    d16 = lax.broadcast_in_dim(d, (_LANES,), ())

    parts = _sc_scatter(L, d16, w16, rw16)
    return _tc_norm_mul(x, parts)


# speedup vs baseline: 1.0559x; 1.0164x over previous
import functools

import jax
import jax.numpy as jnp
from jax import lax
from jax.experimental import pallas as pl
from jax.experimental.pallas import tpu as pltpu
from jax.experimental.pallas import tpu_sc as plsc

_K = 16
_LANES = 16
_NC = 2
_NS = 16
_PAD = 256
_UNROLL = 2


def _sc_scatter_body(L, p_hbm, out_hbm, p_v, wm_v, hc_v):
    n_chunks = L // (_NC * _NS * _LANES)
    cid = lax.axis_index("c")
    sid = lax.axis_index("s")
    wid = cid * _NS + sid

    pltpu.sync_copy(p_hbm, p_v)

    iota_i = jax.lax.iota(jnp.int32, _LANES)
    iota_f = iota_i.astype(jnp.float32)
    zf = jnp.zeros((_LANES,), jnp.float32)
    dvec = p_v[2 * _K]

    Lp = L + _PAD

    def zbody(i, _):
        off = i * (4 * _LANES)
        for u in range(4):
            wm_v[pl.ds(off + u * _LANES, _LANES)] = zf
            hc_v[pl.ds(off + u * _LANES, _LANES)] = zf
        return 0

    lax.fori_loop(0, Lp // (4 * _LANES), zbody, 0, unroll=False)

    l_base = (wid * (L // (_NC * _NS))).astype(jnp.float32)
    fmax = float(L - 1)

    blo_w = blo_h = bhi_w = bhi_h = zf

    def chunk(lvec, c_lo, c_hi, wk, rwk):
        centers = lvec + dvec
        p = centers + wk
        p_cl = jnp.minimum(jnp.maximum(p, 0.0), fmax)
        pf_i = p_cl.astype(jnp.int32)
        frac = p_cl - pf_i.astype(jnp.float32)
        omf = 1.0 - frac
        m_lo = p <= 0.0
        m_hi = p >= fmax
        m_in = jnp.logical_not(jnp.logical_or(m_lo, m_hi))
        pc_i = pf_i + 1
        plsc.addupdate_scatter(wm_v, [pf_i], omf * rwk, mask=m_in)
        plsc.addupdate_scatter(wm_v, [pc_i], frac * rwk, mask=m_in)
        plsc.addupdate_scatter(hc_v, [pf_i], omf, mask=m_in)
        plsc.addupdate_scatter(hc_v, [pc_i], frac, mask=m_in)
        c_lo = c_lo + jnp.where(m_lo, 1.0, 0.0)
        c_hi = c_hi + jnp.where(m_hi, 1.0, 0.0)
        return c_lo, c_hi

    _GRP = 4
    for k in range(0, _K, _GRP):
        wks = [p_v[k + g] for g in range(_GRP)]
        rwks = [p_v[_K + k + g] for g in range(_GRP)]

        def body(j, c, wks=wks, rwks=rwks):
            lvec = c[0]
            cs = list(c[1:])
            for _ in range(_UNROLL):
                for g in range(_GRP):
                    cs[2 * g], cs[2 * g + 1] = chunk(
                        lvec, cs[2 * g], cs[2 * g + 1], wks[g], rwks[g])
                lvec = lvec + float(_LANES)
            return (lvec, *cs)

        out_c = lax.fori_loop(0, n_chunks // _UNROLL, body,
                              ((l_base + iota_f),) + tuple([zf] * (2 * _GRP)))
        for g in range(_GRP):
            c_lo, c_hi = out_c[1 + 2 * g], out_c[2 + 2 * g]
            blo_w = blo_w + c_lo * rwks[g]
            blo_h = blo_h + c_lo
            bhi_w = bhi_w + c_hi * rwks[g]
            bhi_h = bhi_h + c_hi

    col_lo = iota_i + L
    col_hi = iota_i + (L + 128)
    plsc.addupdate_scatter(wm_v, [col_lo], blo_w)
    plsc.addupdate_scatter(hc_v, [col_lo], blo_h)
    plsc.addupdate_scatter(wm_v, [col_hi], bhi_w)
    plsc.addupdate_scatter(hc_v, [col_hi], bhi_h)

    pltpu.sync_copy(wm_v, out_hbm.at[wid, 0])
    pltpu.sync_copy(hc_v, out_hbm.at[wid, 1])


def _sc_scatter(L, params):
    Lp = L + _PAD
    mesh = plsc.VectorSubcoreMesh(core_axis_name="c", subcore_axis_name="s")
    fn = functools.partial(
        pl.kernel,
        out_type=jax.ShapeDtypeStruct((_NC * _NS, 2, Lp), jnp.float32),
        mesh=mesh,
        compiler_params=pltpu.CompilerParams(needs_layout_passes=False),
        scratch_types=[
            pltpu.VMEM((2 * _K + 2, _LANES), jnp.float32),
            pltpu.VMEM((Lp,), jnp.float32),
            pltpu.VMEM((Lp,), jnp.float32),
        ],
    )(functools.partial(_sc_scatter_body, L))
    return fn(params)


def _tc_params_body(K, bd_ref, bs_ref, bbf_ref, bbb_ref, bst_ref, baf_ref,
                    bab_ref, bias_ref, p_ref):
    def sp(v):
        return jax.nn.softplus(v)

    d = jnp.clip(bd_ref[0, 0], -32.0, 32.0)
    stride = sp(bs_ref[0, 0])
    beta_fwd = sp(bbf_ref[0, 0])
    beta_bwd = sp(bbb_ref[0, 0])
    strength = sp(bst_ref[0, 0])
    alpha_fwd = sp(baf_ref[0, 0])
    alpha_bwd = sp(bab_ref[0, 0])
    k2 = lax.broadcasted_iota(jnp.int32, (K, _LANES), 0).astype(jnp.float32) - (K // 2)
    k_abs = jnp.abs(k2)
    w16 = jnp.where(k2 >= 0, (k_abs ** beta_fwd) * stride,
                    -((k_abs ** beta_bwd) * stride))
    envelope = jnp.where(k2 >= 0, strength / (1.0 + k_abs) ** alpha_fwd,
                         strength / (1.0 + k_abs) ** alpha_bwd)
    rw16 = envelope * (1.0 + jnp.tanh(bias_ref[...]))
    p_ref[0:K, :] = w16
    p_ref[K:2 * K, :] = rw16
    p_ref[2 * K:2 * K + 2, :] = jnp.full((2, _LANES), d, jnp.float32)


def _tc_params(base_deformation, base_stride, base_beta_fwd, base_beta_bwd,
               base_strength, base_alpha_fwd, base_alpha_bwd, sample_bias):
    K = _K
    scal = pl.BlockSpec(memory_space=pltpu.SMEM)
    scalars = [s.reshape(1, 1) for s in (
        base_deformation, base_stride, base_beta_fwd, base_beta_bwd,
        base_strength, base_alpha_fwd, base_alpha_bwd)]
    bias2 = sample_bias.reshape(K, 1)
    return pl.pallas_call(
        functools.partial(_tc_params_body, K),
        in_specs=[scal] * 7 + [pl.BlockSpec((K, 1), lambda: (0, 0))],
        out_specs=pl.BlockSpec((2 * K + 2, _LANES), lambda: (0, 0)),
        out_shape=jax.ShapeDtypeStruct((2 * K + 2, _LANES), jnp.float32),
    )(*scalars, bias2)


def _tc_norm_mul_body(L, parts_ref, x_ref, o_ref, norm_ref):
    @pl.when(pl.program_id(0) == 0)
    def _():
        s = jnp.sum(parts_ref[...], axis=0)
        pw = s[0:1, :]
        ph = s[1:2, :]
        b0w = jnp.sum(pw[:, L:L + 16])
        b0h = jnp.sum(ph[:, L:L + 16])
        bLw = jnp.sum(pw[:, L + 128:L + 144])
        bLh = jnp.sum(ph[:, L + 128:L + 144])
        col = lax.broadcasted_iota(jnp.int32, (1, L), 1)
        wm = pw[:, :L] + jnp.where(col == 0, b0w, 0.0) + jnp.where(col == L - 1, bLw, 0.0)
        hc = ph[:, :L] + jnp.where(col == 0, b0h, 0.0) + jnp.where(col == L - 1, bLh, 0.0)
        avg = wm / jnp.maximum(hc, 1e-6)
        norm_ref[...] = avg / jnp.maximum(jnp.sum(avg, axis=1, keepdims=True), 1e-6)

    o_ref[...] = x_ref[...] * norm_ref[...]


def _tc_norm_mul(x, parts):
    B, L = x.shape
    Lp = L + _PAD
    rows = 32
    grid = (B // rows,)
    return pl.pallas_call(
        functools.partial(_tc_norm_mul_body, L),
        grid=grid,
        in_specs=[
            pl.BlockSpec((_NC * _NS, 2, Lp), lambda i: (0, 0, 0)),
            pl.BlockSpec((rows, L), lambda i: (i, 0)),
        ],
        out_specs=pl.BlockSpec((rows, L), lambda i: (i, 0)),
        out_shape=jax.ShapeDtypeStruct((B, L), x.dtype),
        scratch_shapes=[pltpu.VMEM((1, L), jnp.float32)],
    )(parts, x)


def kernel(x, base_deformation, base_stride, base_beta_fwd, base_beta_bwd,
           base_strength, base_alpha_fwd, base_alpha_bwd, sample_bias):
    B, L = x.shape
    params = _tc_params(base_deformation, base_stride, base_beta_fwd,
                        base_beta_bwd, base_strength, base_alpha_fwd,
                        base_alpha_bwd, sample_bias)
    parts = _sc_scatter(L, params)
    return _tc_norm_mul(x, parts)


# speedup vs baseline: 1.0788x; 1.0216x over previous
import functools

import jax
import jax.numpy as jnp
from jax import lax
from jax.experimental import pallas as pl
from jax.experimental.pallas import tpu as pltpu
from jax.experimental.pallas import tpu_sc as plsc

_K = 16
_LANES = 16
_NC = 2
_NS = 16
_PAD = 256
_UNROLL = 2


def _sc_scatter_body(L, p_hbm, out_hbm, p_v, wm_v, hc_v):
    n_chunks = L // (_NC * _NS * _LANES)
    cid = lax.axis_index("c")
    sid = lax.axis_index("s")
    wid = cid * _NS + sid

    pltpu.sync_copy(p_hbm, p_v)

    iota_i = jax.lax.iota(jnp.int32, _LANES)
    iota_f = iota_i.astype(jnp.float32)
    zf = jnp.zeros((_LANES,), jnp.float32)
    dvec = p_v[2 * _K]

    Lp = L + _PAD

    def zbody(i, _):
        off = i * (4 * _LANES)
        for u in range(4):
            wm_v[pl.ds(off + u * _LANES, _LANES)] = zf
            hc_v[pl.ds(off + u * _LANES, _LANES)] = zf
        return 0

    lax.fori_loop(0, Lp // (4 * _LANES), zbody, 0, unroll=False)

    l_base = (wid * (L // (_NC * _NS))).astype(jnp.float32)
    fmax = float(L - 1)

    blo_w = blo_h = bhi_w = bhi_h = zf

    def chunk(lvec, c_lo, c_hi, wk, rwk):
        centers = lvec + dvec
        p = centers + wk
        p_cl = jnp.minimum(jnp.maximum(p, 0.0), fmax)
        pf_i = p_cl.astype(jnp.int32)
        frac = p_cl - pf_i.astype(jnp.float32)
        omf = 1.0 - frac
        m_lo = p <= 0.0
        m_hi = p >= fmax
        m_in = jnp.logical_not(jnp.logical_or(m_lo, m_hi))
        pc_i = pf_i + 1
        plsc.addupdate_scatter(wm_v, [pf_i], omf * rwk, mask=m_in)
        plsc.addupdate_scatter(wm_v, [pc_i], frac * rwk, mask=m_in)
        plsc.addupdate_scatter(hc_v, [pf_i], omf, mask=m_in)
        plsc.addupdate_scatter(hc_v, [pc_i], frac, mask=m_in)
        c_lo = c_lo + jnp.where(m_lo, 1.0, 0.0)
        c_hi = c_hi + jnp.where(m_hi, 1.0, 0.0)
        return c_lo, c_hi

    _GRP = 4
    for k in range(0, _K, _GRP):
        wks = [p_v[k + g] for g in range(_GRP)]
        rwks = [p_v[_K + k + g] for g in range(_GRP)]

        def body(j, c, wks=wks, rwks=rwks):
            lvec = c[0]
            cs = list(c[1:])
            for _ in range(_UNROLL):
                for g in range(_GRP):
                    cs[2 * g], cs[2 * g + 1] = chunk(
                        lvec, cs[2 * g], cs[2 * g + 1], wks[g], rwks[g])
                lvec = lvec + float(_LANES)
            return (lvec, *cs)

        out_c = lax.fori_loop(0, n_chunks // _UNROLL, body,
                              ((l_base + iota_f),) + tuple([zf] * (2 * _GRP)))
        for g in range(_GRP):
            c_lo, c_hi = out_c[1 + 2 * g], out_c[2 + 2 * g]
            blo_w = blo_w + c_lo * rwks[g]
            blo_h = blo_h + c_lo
            bhi_w = bhi_w + c_hi * rwks[g]
            bhi_h = bhi_h + c_hi

    col_lo = iota_i + L
    col_hi = iota_i + (L + 128)
    plsc.addupdate_scatter(wm_v, [col_lo], blo_w)
    plsc.addupdate_scatter(hc_v, [col_lo], blo_h)
    plsc.addupdate_scatter(wm_v, [col_hi], bhi_w)
    plsc.addupdate_scatter(hc_v, [col_hi], bhi_h)

    pltpu.sync_copy(wm_v, out_hbm.at[wid, 0])
    pltpu.sync_copy(hc_v, out_hbm.at[wid, 1])


def _sc_scatter(L, params):
    Lp = L + _PAD
    mesh = plsc.VectorSubcoreMesh(core_axis_name="c", subcore_axis_name="s")
    fn = functools.partial(
        pl.kernel,
        out_type=jax.ShapeDtypeStruct((_NC * _NS, 2, Lp), jnp.float32),
        mesh=mesh,
        compiler_params=pltpu.CompilerParams(needs_layout_passes=False),
        scratch_types=[
            pltpu.VMEM((2 * _K + 2, _LANES), jnp.float32),
            pltpu.VMEM((Lp,), jnp.float32),
            pltpu.VMEM((Lp,), jnp.float32),
        ],
    )(functools.partial(_sc_scatter_body, L))
    return fn(params)


def _mk_params(base_deformation, base_stride, base_beta_fwd, base_beta_bwd,
               base_strength, base_alpha_fwd, base_alpha_bwd, sample_bias):
    K = _K
    R = 2 * K + 2
    d = jnp.clip(base_deformation, -32.0, 32.0)
    stride = jax.nn.softplus(base_stride)
    beta_fwd = jax.nn.softplus(base_beta_fwd)
    beta_bwd = jax.nn.softplus(base_beta_bwd)
    strength = jax.nn.softplus(base_strength)
    alpha_fwd = jax.nn.softplus(base_alpha_fwd)
    alpha_bwd = jax.nn.softplus(base_alpha_bwd)
    r = lax.broadcasted_iota(jnp.int32, (R, _LANES), 0)
    kk = jnp.where(r < K, r, r - K).astype(jnp.float32) - (K // 2)
    k_abs = jnp.abs(kk)
    fwd = kk >= 0
    w_val = jnp.where(fwd, (k_abs ** beta_fwd) * stride,
                      -((k_abs ** beta_bwd) * stride))
    envelope = jnp.where(fwd, strength / (1.0 + k_abs) ** alpha_fwd,
                         strength / (1.0 + k_abs) ** alpha_bwd)
    bias_r = lax.broadcast_in_dim(
        jnp.concatenate([sample_bias, sample_bias, jnp.zeros(2, jnp.float32)]),
        (R, _LANES), (0,))
    rw_val = envelope * (1.0 + jnp.tanh(bias_r))
    out = jnp.where(r < K, w_val, jnp.where(r < 2 * K, rw_val, d))
    return out


def _tc_norm_mul_body(L, parts_ref, x_ref, o_ref, norm_ref):
    @pl.when(pl.program_id(0) == 0)
    def _():
        s = jnp.sum(parts_ref[...], axis=0)
        pw = s[0:1, :]
        ph = s[1:2, :]
        b0w = jnp.sum(pw[:, L:L + 16])
        b0h = jnp.sum(ph[:, L:L + 16])
        bLw = jnp.sum(pw[:, L + 128:L + 144])
        bLh = jnp.sum(ph[:, L + 128:L + 144])
        col = lax.broadcasted_iota(jnp.int32, (1, L), 1)
        wm = pw[:, :L] + jnp.where(col == 0, b0w, 0.0) + jnp.where(col == L - 1, bLw, 0.0)
        hc = ph[:, :L] + jnp.where(col == 0, b0h, 0.0) + jnp.where(col == L - 1, bLh, 0.0)
        avg = wm / jnp.maximum(hc, 1e-6)
        norm_ref[...] = avg / jnp.maximum(jnp.sum(avg, axis=1, keepdims=True), 1e-6)

    o_ref[...] = x_ref[...] * norm_ref[...]


def _tc_norm_mul(x, parts):
    B, L = x.shape
    Lp = L + _PAD
    rows = 32
    grid = (B // rows,)
    return pl.pallas_call(
        functools.partial(_tc_norm_mul_body, L),
        grid=grid,
        in_specs=[
            pl.BlockSpec((_NC * _NS, 2, Lp), lambda i: (0, 0, 0)),
            pl.BlockSpec((rows, L), lambda i: (i, 0)),
        ],
        out_specs=pl.BlockSpec((rows, L), lambda i: (i, 0)),
        out_shape=jax.ShapeDtypeStruct((B, L), x.dtype),
        scratch_shapes=[pltpu.VMEM((1, L), jnp.float32)],
    )(parts, x)


def kernel(x, base_deformation, base_stride, base_beta_fwd, base_beta_bwd,
           base_strength, base_alpha_fwd, base_alpha_bwd, sample_bias):
    B, L = x.shape
    params = _mk_params(base_deformation, base_stride, base_beta_fwd,
                        base_beta_bwd, base_strength, base_alpha_fwd,
                        base_alpha_bwd, sample_bias)
    parts = _sc_scatter(L, params)
    return _tc_norm_mul(x, parts)
